# baseline (speedup 1.0000x reference)
import functools

import jax
import jax.numpy as jnp
from jax import lax
from jax.experimental import pallas as pl
from jax.experimental.pallas import tpu as pltpu

_BLK = 8192
_LANE = 128
_SUB = 8
_TILE = _SUB * _LANE


def _zero_argmax_body(x_ref, zero_ref, idx_ref, rmax_ref, ridx_ref, *, nv, v):
    j = pl.program_id(0)

    @pl.when(j == 0)
    def _():
        rmax_ref[...] = jnp.full(rmax_ref.shape, -jnp.inf, rmax_ref.dtype)
        ridx_ref[...] = jnp.zeros(ridx_ref.shape, ridx_ref.dtype)

    zero_ref[...] = jnp.zeros(zero_ref.shape, zero_ref.dtype)

    x = x_ref[...]
    col = lax.broadcasted_iota(jnp.int32, x.shape, 1) + j * x.shape[1]
    x = jnp.where(col < v, x, -jnp.inf)
    m = jnp.max(x, axis=1, keepdims=True)
    lidx = jnp.min(jnp.where(x == m, col, v), axis=1, keepdims=True)
    better = m > rmax_ref[...]
    ridx_ref[...] = jnp.where(better, lidx, ridx_ref[...])
    rmax_ref[...] = jnp.where(better, m, rmax_ref[...])

    @pl.when(j == nv - 1)
    def _():
        idx_ref[...] = ridx_ref[...]


def _scatter_body(off_ref, a_ref, base_ref, zero_ref, out_ref, p_ref, sem,
                  *, b):
    del zero_ref
    sub = lax.broadcasted_iota(jnp.int32, p_ref.shape, 0) % _SUB
    lane = lax.broadcasted_iota(jnp.int32, p_ref.shape, 1)
    target = base_ref[...] + sub * _LANE + lane
    hit = (target == a_ref[..., 0:1]) | (target == a_ref[..., 1:2])
    hit = hit | (target == a_ref[..., 2:3])
    p_ref[...] = hit.astype(p_ref.dtype)

    copies = [
        pltpu.make_async_copy(
            p_ref.at[pl.ds(r * _SUB, _SUB), :],
            out_ref.at[pl.ds(pl.multiple_of(off_ref[r], _SUB), _SUB), :],
            sem,
        )
        for r in range(b)
    ]
    for c in copies:
        c.start()
    for c in copies:
        c.wait()


def kernel(logits):
    b, v = logits.shape
    nv = pl.cdiv(v, _BLK)
    assert v >= _TILE and (b * v) % _TILE == 0

    zeros, idx = pl.pallas_call(
        functools.partial(_zero_argmax_body, nv=nv, v=v),
        grid=(nv,),
        in_specs=[pl.BlockSpec((b, _BLK), lambda i: (0, i))],
        out_specs=[
            pl.BlockSpec((b, _BLK), lambda i: (0, i)),
            pl.BlockSpec((b, 1), lambda i: (0, 0)),
        ],
        out_shape=[
            jax.ShapeDtypeStruct((b, v), logits.dtype),
            jax.ShapeDtypeStruct((b, 1), jnp.int32),
        ],
        scratch_shapes=[
            pltpu.VMEM((b, 1), jnp.float32),
            pltpu.VMEM((b, 1), jnp.int32),
        ],
    )(logits)

    flat = idx[:, 0] + jnp.arange(b, dtype=jnp.int32) * v
    tile = flat // _TILE
    off = tile * _SUB
    prev_f = jnp.concatenate([flat[:1], flat[:-1]])
    next_f = jnp.concatenate([flat[1:], flat[-1:]])
    cand = jnp.stack([prev_f, flat, next_f], axis=1)
    cand_rep = jnp.repeat(cand, _SUB, axis=0)
    base_rep = jnp.repeat((tile * _TILE)[:, None], _SUB, axis=0)

    zeros_flat = zeros.reshape(b * v // _LANE, _LANE)

    grid_spec = pltpu.PrefetchScalarGridSpec(
        num_scalar_prefetch=1,
        grid=(1,),
        in_specs=[
            pl.BlockSpec((_SUB * b, 3), lambda i, off_pref: (0, 0)),
            pl.BlockSpec((_SUB * b, 1), lambda i, off_pref: (0, 0)),
            pl.BlockSpec(memory_space=pl.ANY),
        ],
        out_specs=pl.BlockSpec(memory_space=pl.ANY),
        scratch_shapes=[
            pltpu.VMEM((_SUB * b, _LANE), logits.dtype),
            pltpu.SemaphoreType.DMA,
        ],
    )
    out = pl.pallas_call(
        functools.partial(_scatter_body, b=b),
        grid_spec=grid_spec,
        out_shape=jax.ShapeDtypeStruct(zeros_flat.shape, logits.dtype),
        input_output_aliases={3: 0},
    )(off, cand_rep, base_rep, zeros_flat)
    return out.reshape(b, v)
